# baseline (device time: 224132 ns/iter reference)
import jax
import jax.numpy as jnp
from jax import lax
from jax.experimental import pallas as pl
from jax.experimental.pallas import tpu as pltpu

N_DEV = 8
CAP = 640


def _a2a_pallas(send_blocks, my_counts):
    n, cap, d_model = send_blocks.shape

    def body(send_ref, cnt_ref, recv_ref, cnt_out_ref,
             send_sems, recv_sems, csend_sems, crecv_sems):
        me = lax.axis_index("i")

        barrier_sem = pltpu.get_barrier_semaphore()
        for dd in range(1, N_DEV):
            t = lax.rem(me + dd, N_DEV)
            pl.semaphore_signal(barrier_sem, inc=1, device_id=(t,),
                                device_id_type=pl.DeviceIdType.MESH)
        pl.semaphore_wait(barrier_sem, N_DEV - 1)

        rdmas = []
        for dd in range(1, N_DEV):
            t = lax.rem(me + dd, N_DEV)
            rdma = pltpu.make_async_remote_copy(
                src_ref=send_ref.at[dd - 1],
                dst_ref=recv_ref.at[dd - 1],
                send_sem=send_sems.at[dd - 1],
                recv_sem=recv_sems.at[dd - 1],
                device_id=(t,),
                device_id_type=pl.DeviceIdType.MESH,
            )
            rdma.start()
            crdma = pltpu.make_async_remote_copy(
                src_ref=cnt_ref,
                dst_ref=cnt_out_ref.at[pl.ds(dd - 1, 1)],
                send_sem=csend_sems.at[dd - 1],
                recv_sem=crecv_sems.at[dd - 1],
                device_id=(t,),
                device_id_type=pl.DeviceIdType.MESH,
            )
            crdma.start()
            rdmas.append((rdma, crdma))

        for rdma, crdma in rdmas:
            rdma.wait()
            crdma.wait()

    return pl.pallas_call(
        body,
        out_shape=[
            jax.ShapeDtypeStruct((N_DEV - 1, cap, d_model), send_blocks.dtype),
            jax.ShapeDtypeStruct((N_DEV - 1, 128), jnp.int32),
        ],
        in_specs=[
            pl.BlockSpec(memory_space=pltpu.VMEM),
            pl.BlockSpec(memory_space=pltpu.VMEM),
        ],
        out_specs=[
            pl.BlockSpec(memory_space=pltpu.VMEM),
            pl.BlockSpec(memory_space=pltpu.VMEM),
        ],
        scratch_shapes=[
            pltpu.SemaphoreType.DMA((N_DEV - 1,)),
            pltpu.SemaphoreType.DMA((N_DEV - 1,)),
            pltpu.SemaphoreType.DMA((N_DEV - 1,)),
            pltpu.SemaphoreType.DMA((N_DEV - 1,)),
        ],
        compiler_params=pltpu.CompilerParams(collective_id=0),
    )(send_blocks, my_counts)


def kernel(x, dest):
    t_loc, d_model = x.shape
    me = lax.axis_index("i")

    xb = x.astype(jnp.bfloat16)
    order = jnp.argsort(dest, stable=True)
    cnt = jnp.bincount(dest, length=N_DEV).astype(jnp.int32)
    cum = jnp.cumsum(cnt)
    offs = (cum - cnt).astype(jnp.int32)

    tgt = jnp.remainder(me + 1 + jnp.arange(N_DEV), N_DEV)
    k = jnp.arange(CAP)
    row_idx = offs[tgt][:, None] + jnp.minimum(
        k[None, :], jnp.maximum(cnt[tgt][:, None] - 1, 0))
    src_rows = order[jnp.clip(row_idx, 0, t_loc - 1)]
    send_blocks = xb[src_rows.reshape(-1)].reshape(N_DEV, CAP, d_model)
    my_counts = jnp.zeros((1, 128), jnp.int32).at[0, :N_DEV].set(cnt)

    recv_blocks, cnt_rows = _a2a_pallas(send_blocks, my_counts)

    full_blocks = jnp.concatenate([recv_blocks, send_blocks[N_DEV - 1:]], 0)
    full_cnts = jnp.concatenate([cnt_rows, my_counts], 0)
    slot_of_src = jnp.remainder(me - 1 - jnp.arange(N_DEV), N_DEV)
    blocks_by_src = full_blocks[slot_of_src]
    sizes = full_cnts[slot_of_src, me]

    cum_s = jnp.cumsum(sizes)
    offs_s = cum_s - sizes
    j = jnp.arange(t_loc)
    seg = jnp.searchsorted(cum_s, j, side="right")
    flat = seg * CAP + (j - offs_s[seg])
    return blocks_by_src.reshape(N_DEV * CAP, d_model)[flat]


# device time: 212839 ns/iter; 1.0531x vs baseline; 1.0531x over previous
import jax
import jax.numpy as jnp
from jax import lax
from jax.experimental import pallas as pl
from jax.experimental.pallas import tpu as pltpu

N_DEV = 8
CAP = 640
CAP_W = CAP // 8


def _a2av_pallas(xb3, src_rows, cnt_row):
    n_win, _, d_model = xb3.shape
    t_loc = n_win * 8
    send_win = N_DEV * CAP_W
    idx_rows = t_loc // 128

    def body(x_ref, srcrow_ref, cnt_ref, out_ref,
             send_ref, recv_ref, cnt_mat_ref, idx_vmem,
             cnt_smem, idx_smem,
             dsend, drecv, csend, crecv, loc_sem):
        me = lax.axis_index("i")
        iota8 = lax.broadcasted_iota(jnp.int32, (8, d_model), 0)

        barrier_sem = pltpu.get_barrier_semaphore()
        for dd in range(1, N_DEV):
            t = lax.rem(me + dd, N_DEV)
            pl.semaphore_signal(barrier_sem, inc=1, device_id=(t,),
                                device_id_type=pl.DeviceIdType.MESH)
        pl.semaphore_wait(barrier_sem, N_DEV - 1)

        crdmas = []
        for dd in range(1, N_DEV):
            t = lax.rem(me + dd, N_DEV)
            c = pltpu.make_async_remote_copy(
                src_ref=cnt_ref,
                dst_ref=cnt_mat_ref.at[pl.ds(dd - 1, 1)],
                send_sem=csend.at[dd - 1],
                recv_sem=crecv.at[dd - 1],
                device_id=(t,),
                device_id_type=pl.DeviceIdType.MESH,
            )
            c.start()
            crdmas.append(c)

        def gather_it(it, carry):
            acc = jnp.zeros((8, d_model), jnp.bfloat16)
            for u in range(8):
                r = srcrow_ref[it * 8 + u]
                w = x_ref[lax.div(r, 8)]
                sh = lax.rem(u - lax.rem(r, 8) + 8, 8)
                acc = jnp.where(iota8 == u, pltpu.roll(w, sh, 0), acc)
            send_ref[it] = acc
            return carry

        lax.fori_loop(0, send_win, gather_it, 0)

        self_cp = pltpu.make_async_copy(
            send_ref.at[pl.ds((N_DEV - 1) * CAP_W, CAP_W)],
            recv_ref.at[pl.ds((N_DEV - 1) * CAP_W, CAP_W)],
            loc_sem,
        )
        self_cp.start()

        rdmas = []
        for dd in range(1, N_DEV):
            t = lax.rem(me + dd, N_DEV)
            rdma = pltpu.make_async_remote_copy(
                src_ref=send_ref.at[pl.ds((dd - 1) * CAP_W, CAP_W)],
                dst_ref=recv_ref.at[pl.ds((dd - 1) * CAP_W, CAP_W)],
                send_sem=dsend.at[dd - 1],
                recv_sem=drecv.at[dd - 1],
                device_id=(t,),
                device_id_type=pl.DeviceIdType.MESH,
            )
            rdma.start()
            rdmas.append(rdma)

        for c in crdmas:
            c.wait()
        self_cp.wait()
        cp1 = pltpu.make_async_copy(
            cnt_mat_ref, cnt_smem.at[pl.ds(0, N_DEV - 1)], loc_sem)
        cp1.start()
        cp1.wait()
        cp2 = pltpu.make_async_copy(
            cnt_ref, cnt_smem.at[pl.ds(N_DEV - 1, 1)], loc_sem)
        cp2.start()
        cp2.wait()

        slots, cums = [], []
        cum = jnp.int32(0)
        for s in range(N_DEV):
            slot = lax.rem(me + (N_DEV - 1) - s, N_DEV)
            slots.append(slot)
            cum = cum + cnt_smem[slot, me]
            cums.append(cum)

        j_vec = (lax.broadcasted_iota(jnp.int32, (idx_rows, 128), 0) * 128
                 + lax.broadcasted_iota(jnp.int32, (idx_rows, 128), 1))
        seg = jnp.zeros((idx_rows, 128), jnp.int32)
        for s in range(N_DEV - 1):
            seg = seg + (j_vec >= cums[s]).astype(jnp.int32)
        slot_v = jnp.zeros((idx_rows, 128), jnp.int32)
        off_v = jnp.zeros((idx_rows, 128), jnp.int32)
        for s in range(N_DEV):
            m = (seg == s).astype(jnp.int32)
            slot_v = slot_v + m * slots[s]
            off_v = off_v + m * (cums[s] - cnt_smem[slots[s], me])
        idx_vmem[:, :] = slot_v * CAP + (j_vec - off_v)

        cp3 = pltpu.make_async_copy(idx_vmem, idx_smem, loc_sem)
        cp3.start()
        cp3.wait()

        for rdma in rdmas:
            rdma.wait()

        def compact_it(it, carry):
            acc = jnp.zeros((8, d_model), jnp.bfloat16)
            for u in range(8):
                j = it * 8 + u
                f = idx_smem[lax.div(j, 128), lax.rem(j, 128)]
                w = recv_ref[lax.div(f, 8)]
                sh = lax.rem(u - lax.rem(f, 8) + 8, 8)
                acc = jnp.where(iota8 == u, pltpu.roll(w, sh, 0), acc)
            out_ref[it] = acc
            return carry

        lax.fori_loop(0, n_win, compact_it, 0)

    return pl.pallas_call(
        body,
        out_shape=jax.ShapeDtypeStruct((n_win, 8, d_model), jnp.bfloat16),
        in_specs=[
            pl.BlockSpec(memory_space=pltpu.VMEM),
            pl.BlockSpec(memory_space=pltpu.SMEM),
            pl.BlockSpec(memory_space=pltpu.VMEM),
        ],
        out_specs=pl.BlockSpec(memory_space=pltpu.VMEM),
        scratch_shapes=[
            pltpu.VMEM((send_win, 8, d_model), jnp.bfloat16),
            pltpu.VMEM((send_win, 8, d_model), jnp.bfloat16),
            pltpu.VMEM((N_DEV - 1, 128), jnp.int32),
            pltpu.VMEM((idx_rows, 128), jnp.int32),
            pltpu.SMEM((N_DEV, 128), jnp.int32),
            pltpu.SMEM((idx_rows, 128), jnp.int32),
            pltpu.SemaphoreType.DMA((N_DEV - 1,)),
            pltpu.SemaphoreType.DMA((N_DEV - 1,)),
            pltpu.SemaphoreType.DMA((N_DEV - 1,)),
            pltpu.SemaphoreType.DMA((N_DEV - 1,)),
            pltpu.SemaphoreType.DMA,
        ],
        compiler_params=pltpu.CompilerParams(collective_id=0),
    )(xb3, src_rows, cnt_row)


def kernel(x, dest):
    t_loc, d_model = x.shape
    me = lax.axis_index("i")

    xb = x.astype(jnp.bfloat16)
    order = jnp.argsort(dest, stable=True)
    oh = (dest[:, None] == jnp.arange(N_DEV)[None, :]).astype(jnp.int32)
    cnt = oh.sum(axis=0).astype(jnp.int32)
    cum = jnp.cumsum(cnt)
    offs = (cum - cnt).astype(jnp.int32)

    tgt = jnp.remainder(me + 1 + jnp.arange(N_DEV), N_DEV)
    k = jnp.arange(CAP)
    row_idx = offs[tgt][:, None] + jnp.minimum(
        k[None, :], jnp.maximum(cnt[tgt][:, None] - 1, 0))
    src_rows = order[jnp.clip(row_idx, 0, t_loc - 1)].reshape(-1)
    src_rows = src_rows.astype(jnp.int32)
    cnt_row = jnp.pad(cnt, (0, 128 - N_DEV)).reshape(1, 128)

    out3 = _a2av_pallas(xb.reshape(t_loc // 8, 8, d_model), src_rows, cnt_row)
    return out3.reshape(t_loc, d_model)
